# Initial kernel scaffold; baseline (speedup 1.0000x reference)
import jax
import jax.numpy as jnp
from jax.experimental import pallas as pl


def kernel(x, edge_index, edge_attr, u, batch, m1_W0, m1_b0, m1_W1, m1_b1, m1_W2, m1_b2, m1_g, m1_beta, m1_W3, m1_b3, m2_W0, m2_b0, m2_W1, m2_b1, m2_W2, m2_b2, m2_g, m2_beta, m2_W3, m2_b3):
    raise NotImplementedError("write your pallas kernel here")


# speedup vs baseline: 3.2749x; 3.2749x over previous
import functools

import jax
import jax.numpy as jnp
from jax import lax
from jax.experimental import pallas as pl
from jax.experimental.pallas import tpu as pltpu
from jax.experimental.pallas import tpu_sc as plsc

NC, NS = 2, 16
NW = NC * NS
CH = 125
GRP = 4
ROWS = CH * GRP
F32 = jnp.float32


def _sc_gather(x, idx2d):
    n, h = x.shape
    e = idx2d.size
    n_outer = e // (NW * ROWS)
    mesh = plsc.VectorSubcoreMesh(core_axis_name="c", subcore_axis_name="s",
                                  num_cores=NC, num_subcores=NS)

    @functools.partial(
        pl.kernel,
        out_type=jax.ShapeDtypeStruct((e, h), F32),
        mesh=mesh,
        scratch_types=[
            pltpu.VMEM((GRP, CH), jnp.int32),
            pltpu.VMEM((ROWS, h), F32),
            pltpu.SemaphoreType.DMA,
        ],
        compiler_params=pltpu.CompilerParams(use_tc_tiling_on_sc=False),
    )
    def k(x_hbm, idx_hbm, out_hbm, idx_v, buf, sem):
        wid = lax.axis_index("s") * NC + lax.axis_index("c")
        row0 = wid * (n_outer * GRP)

        def outer(o, carry):
            pltpu.sync_copy(idx_hbm.at[pl.ds(row0 + o * GRP, GRP)], idx_v)
            descs = [
                pltpu.async_copy(x_hbm.at[idx_v.at[j]],
                                 buf.at[pl.ds(j * CH, CH)], sem)
                for j in range(GRP)
            ]
            for d in descs:
                d.wait()
            pltpu.sync_copy(buf, out_hbm.at[pl.ds((row0 + o * GRP) * CH, ROWS)])
            return carry

        lax.fori_loop(0, n_outer, outer, 0)

    return k(x, idx2d)


def _sc_scatter(ye, col2d, n):
    e, h = ye.shape
    grp = 2
    rows = CH * grp
    n_outer = e // (NW * rows)
    rows_per_tile = n // NS
    mesh = plsc.VectorSubcoreMesh(core_axis_name="c", subcore_axis_name="s",
                                  num_cores=NC, num_subcores=NS)

    @functools.partial(
        pl.kernel,
        out_type=jax.ShapeDtypeStruct((NC, n, h), F32),
        mesh=mesh,
        scratch_types=[
            pltpu.VMEM((grp, CH), jnp.int32),
            pltpu.VMEM((rows, h), F32),
            pltpu.VMEM_SHARED((n, h), F32),
        ],
        compiler_params=pltpu.CompilerParams(use_tc_tiling_on_sc=False),
    )
    def k(ye_hbm, col_hbm, out_hbm, col_v, buf, acc):
        cid = lax.axis_index("c")
        sid = lax.axis_index("s")

        def zrow(i, carry):
            for j in range(h // 16):
                buf[i, pl.ds(j * 16, 16)] = jnp.zeros((16,), F32)
            return carry
        lax.fori_loop(0, CH, zrow, 0)
        for r in range(rows_per_tile // CH):
            pltpu.sync_copy(buf.at[pl.ds(0, CH)],
                            acc.at[pl.ds(sid * rows_per_tile + r * CH, CH)])
        plsc.subcore_barrier()

        e0 = (cid * NS + sid) * (n_outer * rows)
        row0 = e0 // CH

        def outer(o, carry):
            pltpu.sync_copy(col_hbm.at[pl.ds(row0 + o * grp, grp)], col_v)
            pltpu.sync_copy(ye_hbm.at[pl.ds(e0 + o * rows, rows)], buf)
            for j in range(grp):
                pltpu.sync_copy(buf.at[pl.ds(j * CH, CH)],
                                acc.at[col_v.at[j]], add=True)
            return carry

        lax.fori_loop(0, n_outer, outer, 0)
        plsc.subcore_barrier()
        pltpu.sync_copy(acc.at[pl.ds(sid * rows_per_tile, rows_per_tile)],
                        out_hbm.at[cid, pl.ds(sid * rows_per_tile, rows_per_tile)])

    return k(ye, col2d)


def _dot(a, b):
    return jnp.dot(a, b, preferred_element_type=F32)


def _edge_mlp(xg, ea, w0x, w0e, b0, w1, b1, w2, b2, g, beta, w3, b3, blk):
    e, h = xg.shape
    d = w1.shape[0]

    def body(xg_r, ea_r, w0x_r, w0e_r, b0_r, w1_r, b1_r, w2_r, b2_r, g_r,
             beta_r, w3_r, b3_r, o_r):
        hh = jnp.maximum(_dot(xg_r[...], w0x_r[...])
                         + _dot(ea_r[...], w0e_r[...]) + b0_r[...], 0.0)
        hh = jnp.maximum(_dot(hh, w1_r[...]) + b1_r[...], 0.0)
        hh = jnp.maximum(_dot(hh, w2_r[...]) + b2_r[...], 0.0)
        mu = jnp.mean(hh, axis=-1, keepdims=True)
        var = jnp.mean(jnp.square(hh - mu), axis=-1, keepdims=True)
        hh = (hh - mu) * lax.rsqrt(var + 1e-5) * g_r[...] + beta_r[...]
        o_r[...] = _dot(hh, w3_r[...]) + b3_r[...]

    full = lambda arr: pl.BlockSpec(arr.shape, lambda i: (0,) * arr.ndim)
    return pl.pallas_call(
        body,
        grid=(e // blk,),
        in_specs=[
            pl.BlockSpec((blk, h), lambda i: (i, 0)),
            pl.BlockSpec((blk, h), lambda i: (i, 0)),
            full(w0x), full(w0e), full(b0), full(w1), full(b1), full(w2),
            full(b2), full(g), full(beta), full(w3), full(b3),
        ],
        out_specs=pl.BlockSpec((blk, h), lambda i: (i, 0)),
        out_shape=jax.ShapeDtypeStruct((e, h), F32),
    )(xg, ea, w0x, w0e, b0, w1, b1, w2, b2, g, beta, w3, b3)


def _node_mlp(x, parts, u, batch2, w0a, w0b, w0c, b0, w1, b1, w2, b2, g, beta,
              w3, b3, blk):
    n, h = x.shape
    nb = u.shape[0]

    def body(x_r, p_r, u_r, bt_r, w0a_r, w0b_r, w0c_r, b0_r, w1_r, b1_r, w2_r,
             b2_r, g_r, beta_r, w3_r, b3_r, o_r):
        xv = x_r[...]
        p = p_r[...]
        agg = p[0] + p[1]
        bt = bt_r[...]
        oh = (bt == lax.broadcasted_iota(jnp.int32, (blk, nb), 1)).astype(F32)
        ub = _dot(oh, u_r[...])
        hh = jnp.maximum(_dot(xv, w0a_r[...]) + _dot(agg, w0b_r[...])
                         + _dot(ub, w0c_r[...]) + b0_r[...], 0.0)
        hh = jnp.maximum(_dot(hh, w1_r[...]) + b1_r[...], 0.0)
        hh = jnp.maximum(_dot(hh, w2_r[...]) + b2_r[...], 0.0)
        mu = jnp.mean(hh, axis=-1, keepdims=True)
        var = jnp.mean(jnp.square(hh - mu), axis=-1, keepdims=True)
        hh = (hh - mu) * lax.rsqrt(var + 1e-5) * g_r[...] + beta_r[...]
        o_r[...] = xv + _dot(hh, w3_r[...]) + b3_r[...]

    full = lambda arr: pl.BlockSpec(arr.shape, lambda i: (0,) * arr.ndim)
    return pl.pallas_call(
        body,
        grid=(n // blk,),
        in_specs=[
            pl.BlockSpec((blk, h), lambda i: (i, 0)),
            pl.BlockSpec((NC, blk, h), lambda i: (0, i, 0)),
            full(u),
            pl.BlockSpec((blk, 1), lambda i: (i, 0)),
            full(w0a), full(w0b), full(w0c), full(b0), full(w1), full(b1),
            full(w2), full(b2), full(g), full(beta), full(w3), full(b3),
        ],
        out_specs=pl.BlockSpec((blk, h), lambda i: (i, 0)),
        out_shape=jax.ShapeDtypeStruct((n, h), F32),
    )(x, parts, u, batch2, w0a, w0b, w0c, b0, w1, b1, w2, b2, g, beta, w3, b3)


def kernel(x, edge_index, edge_attr, u, batch,
           m1_W0, m1_b0, m1_W1, m1_b1, m1_W2, m1_b2, m1_g, m1_beta, m1_W3, m1_b3,
           m2_W0, m2_b0, m2_W1, m2_b1, m2_W2, m2_b2, m2_g, m2_beta, m2_W3, m2_b3):
    n, h = x.shape
    e = edge_attr.shape[0]
    row2 = edge_index[0].reshape(e // CH, CH)
    col2 = edge_index[1].reshape(e // CH, CH)

    xg = _sc_gather(x, row2)
    ye = _edge_mlp(xg, edge_attr,
                   m1_W0[:h], m1_W0[h:], m1_b0.reshape(1, -1),
                   m1_W1, m1_b1.reshape(1, -1), m1_W2, m1_b2.reshape(1, -1),
                   m1_g.reshape(1, -1), m1_beta.reshape(1, -1),
                   m1_W3, m1_b3.reshape(1, -1), blk=2560)
    parts = _sc_scatter(ye, col2, n)
    out = _node_mlp(x, parts, u, batch.reshape(n, 1),
                    m2_W0[:h], m2_W0[h:2 * h], m2_W0[2 * h:],
                    m2_b0.reshape(1, -1), m2_W1, m2_b1.reshape(1, -1),
                    m2_W2, m2_b2.reshape(1, -1), m2_g.reshape(1, -1),
                    m2_beta.reshape(1, -1), m2_W3, m2_b3.reshape(1, -1),
                    blk=2000)
    return out
